# baseline (speedup 1.0000x reference)
import functools

import jax
import jax.numpy as jnp
from jax import lax
from jax.experimental import pallas as pl
from jax.experimental.pallas import tpu as pltpu
from jax.experimental.pallas import tpu_sc as plsc


def _logits_kernel(x_ref, wr_ref, br_ref, lt_ref):
    lt = lax.dot_general(wr_ref[...], x_ref[...], (((0,), (1,)), ((), ())),
                         preferred_element_type=jnp.float32)
    lt_ref[...] = lt + br_ref[...]


def _sc_router_body(lt_hbm, ai_hbm, wv_hbm, sched_hbm, nact_hbm,
                    lt_v, act_v, ai_v, wv_v, sched_v, nact_v):
    E, T = lt_v.shape
    L = 16
    n_tok_chunks = T // L
    n_exp_chunks = E // L
    f32 = jnp.float32
    i32 = jnp.int32

    @pl.when((lax.axis_index("c") == 0) & (lax.axis_index("s") == 0))
    def _():
        pltpu.sync_copy(lt_hbm, lt_v)

        lane = lax.iota(i32, L)

        def step(e, carry):
            m1s, a1s, m2s, a2s = carry
            ev = jnp.full((L,), e, i32)
            new = ([], [], [], [])
            for c in range(n_tok_chunks):
                v = lt_v[e, pl.ds(c * L, L)]
                gt1 = v > m1s[c]
                gt2 = v > m2s[c]
                new[2].append(jnp.where(gt1, m1s[c],
                                        jnp.where(gt2, v, m2s[c])))
                new[3].append(jnp.where(gt1, a1s[c],
                                        jnp.where(gt2, ev, a2s[c])))
                new[0].append(jnp.where(gt1, v, m1s[c]))
                new[1].append(jnp.where(gt1, ev, a1s[c]))
            return tuple(tuple(t) for t in new)

        ninf = jnp.full((L,), -jnp.inf, f32)
        zero_i = jnp.zeros((L,), i32)
        m1s, a1s, m2s, a2s = lax.fori_loop(
            0, E, step,
            ((ninf,) * n_tok_chunks, (zero_i,) * n_tok_chunks,
             (ninf,) * n_tok_chunks, (zero_i,) * n_tok_chunks))

        ones = jnp.ones((L,), f32)
        for c in range(n_tok_chunks):
            sl = pl.ds(c * L, L)
            r = jnp.exp(m2s[c] - m1s[c])
            w1 = 1.0 / (1.0 + r)
            ai_v[0, sl] = a1s[c]
            ai_v[1, sl] = a2s[c]
            wv_v[0, sl] = w1
            wv_v[1, sl] = 1.0 - w1

        for c in range(n_exp_chunks):
            act_v[pl.ds(c * L, L)] = jnp.zeros((L,), f32)
        for c in range(n_tok_chunks):
            plsc.store_scatter(act_v, [a1s[c]], ones)
            plsc.store_scatter(act_v, [a2s[c]], ones)

        carry = f32(0.0)
        last_active = i32(0)
        actives, cums = [], []
        for c in range(n_exp_chunks):
            ch = act_v[pl.ds(c * L, L)]
            cs = plsc.cumsum(ch) + carry
            carry = carry + jnp.sum(ch)
            eid = lane + i32(c * L)
            last_active = jnp.maximum(
                last_active, jnp.max(jnp.where(ch > 0.5, eid, -1)))
            actives.append(ch)
            cums.append(cs)

        lastv = jnp.full((L,), 0, i32) + last_active
        for c in range(n_exp_chunks):
            sched_v[pl.ds(c * L, L)] = lastv
        for c in range(n_exp_chunks):
            rnk = (cums[c] - 1.0).astype(i32)
            eid = lane + i32(c * L)
            plsc.store_scatter(sched_v, [rnk], eid, mask=actives[c] > 0.5)

        nact_v[...] = jnp.full((L,), 0, i32) + carry.astype(i32)

        pltpu.sync_copy(ai_v, ai_hbm)
        pltpu.sync_copy(wv_v, wv_hbm)
        pltpu.sync_copy(sched_v, sched_hbm)
        pltpu.sync_copy(nact_v, nact_hbm)


def _expert_kernel(sched_ref, nact_ref, x_ref, w1a_ref, w1b_ref, b1_ref,
                   w2a_ref, w2b_ref, b2_ref, w3a_ref, w3b_ref, b3_ref,
                   ai_ref, wv_ref, out_ref):
    g = pl.program_id(0)
    T = x_ref.shape[0]
    H2 = w1a_ref.shape[2]
    f32 = jnp.float32

    @pl.when(g == 0)
    def _():
        out_ref[...] = jnp.zeros_like(out_ref)

    @pl.when(g < nact_ref[0])
    def _():
        e = sched_ref[g]
        x = x_ref[...]
        h1a = jnp.dot(x, w1a_ref[0], preferred_element_type=f32)
        h1b = jnp.dot(x, w1b_ref[0], preferred_element_type=f32)
        h1a = h1a + b1_ref[pl.ds(e, 1), :H2]
        h1b = h1b + b1_ref[pl.ds(e, 1), H2:]
        h2 = (jnp.dot(h1a, w2a_ref[0], preferred_element_type=f32)
              + jnp.dot(h1b, w2b_ref[0], preferred_element_type=f32))
        h2 = h2 + b2_ref[pl.ds(e, 1), :]
        a = h2 * (1.0 / (1.0 + jnp.exp(-h2)))
        y = (jnp.dot(a[:, :H2], w3a_ref[0], preferred_element_type=f32)
             + jnp.dot(a[:, H2:], w3b_ref[0], preferred_element_type=f32))
        y = y + b3_ref[pl.ds(e, 1), :]

        c = (jnp.where(ai_ref[0:1, :] == e, wv_ref[0:1, :], f32(0.0))
             + jnp.where(ai_ref[1:2, :] == e, wv_ref[1:2, :], f32(0.0)))
        ii = lax.broadcasted_iota(jnp.int32, (T, T), 0)
        jj = lax.broadcasted_iota(jnp.int32, (T, T), 1)
        dc = jnp.where(ii == jj, c, f32(0.0))
        out_ref[...] += jnp.dot(dc, y, preferred_element_type=f32)


def kernel(x, Wr, br, W1, b1, W2, b2, W3, b3):
    B, S, D = x.shape
    E = Wr.shape[1]
    H = W1.shape[2]
    T = B * S
    x2 = x.reshape(T, D)

    logits_t = pl.pallas_call(
        _logits_kernel,
        out_shape=jax.ShapeDtypeStruct((E, T), jnp.float32),
    )(x2, Wr, br.reshape(E, 1))

    sc_router = functools.partial(
        pl.kernel,
        out_type=[
            jax.ShapeDtypeStruct((2, T), jnp.int32),
            jax.ShapeDtypeStruct((2, T), jnp.float32),
            jax.ShapeDtypeStruct((E,), jnp.int32),
            jax.ShapeDtypeStruct((16,), jnp.int32),
        ],
        mesh=plsc.VectorSubcoreMesh(core_axis_name="c", subcore_axis_name="s",
                                    num_cores=1),
        compiler_params=pltpu.CompilerParams(needs_layout_passes=False),
        scratch_types=[
            pltpu.VMEM((E, T), jnp.float32),
            pltpu.VMEM((E,), jnp.float32),
            pltpu.VMEM((2, T), jnp.int32),
            pltpu.VMEM((2, T), jnp.float32),
            pltpu.VMEM((E,), jnp.int32),
            pltpu.VMEM((16,), jnp.int32),
        ],
    )(_sc_router_body)
    ai, wv, sched, nactv = sc_router(logits_t)

    out = pl.pallas_call(
        _expert_kernel,
        grid_spec=pltpu.PrefetchScalarGridSpec(
            num_scalar_prefetch=2,
            grid=(E,),
            in_specs=[
                pl.BlockSpec((T, D), lambda g, s, n: (0, 0)),
                pl.BlockSpec((1, D, H // 2), lambda g, s, n: (s[g], 0, 0)),
                pl.BlockSpec((1, D, H // 2), lambda g, s, n: (s[g], 0, 1)),
                pl.BlockSpec((E, H), lambda g, s, n: (0, 0)),
                pl.BlockSpec((1, H // 2, H), lambda g, s, n: (s[g], 0, 0)),
                pl.BlockSpec((1, H // 2, H), lambda g, s, n: (s[g], 1, 0)),
                pl.BlockSpec((E, H), lambda g, s, n: (0, 0)),
                pl.BlockSpec((1, H // 2, D), lambda g, s, n: (s[g], 0, 0)),
                pl.BlockSpec((1, H // 2, D), lambda g, s, n: (s[g], 1, 0)),
                pl.BlockSpec((E, D), lambda g, s, n: (0, 0)),
                pl.BlockSpec((2, T), lambda g, s, n: (0, 0)),
                pl.BlockSpec((2, T), lambda g, s, n: (0, 0)),
            ],
            out_specs=pl.BlockSpec((T, D), lambda g, s, n: (0, 0)),
        ),
        out_shape=jax.ShapeDtypeStruct((T, D), jnp.float32),
        compiler_params=pltpu.CompilerParams(
            dimension_semantics=("arbitrary",),
        ),
    )(sched, nactv, x2, W1, W1, b1, W2, W2, b2, W3, W3, b3, ai, wv)

    return out.reshape(B, S, D)


# speedup vs baseline: 1.0123x; 1.0123x over previous
import functools

import jax
import jax.numpy as jnp
from jax import lax
from jax.experimental import pallas as pl
from jax.experimental.pallas import tpu as pltpu
from jax.experimental.pallas import tpu_sc as plsc


def _logits_kernel(x_ref, wr_ref, br_ref, lt_ref):
    lt = lax.dot_general(wr_ref[...], x_ref[...], (((0,), (1,)), ((), ())),
                         preferred_element_type=jnp.float32)
    lt_ref[...] = lt + br_ref[...]


def _sc_router_body(lt_hbm, ai_hbm, wv_hbm, sched_hbm, nact_hbm,
                    lt_v, act_v, ai_v, wv_v, sched_v, nact_v):
    E, T = lt_v.shape
    L = 16
    n_tok_chunks = T // L
    n_exp_chunks = E // L
    f32 = jnp.float32
    i32 = jnp.int32

    @pl.when((lax.axis_index("c") == 0) & (lax.axis_index("s") == 0))
    def _():
        pltpu.sync_copy(lt_hbm, lt_v)

        lane = lax.iota(i32, L)

        def step(e, carry):
            m1s, a1s, m2s, a2s = carry
            ev = jnp.full((L,), e, i32)
            new = ([], [], [], [])
            for c in range(n_tok_chunks):
                v = lt_v[e, pl.ds(c * L, L)]
                gt1 = v > m1s[c]
                gt2 = v > m2s[c]
                new[2].append(jnp.where(gt1, m1s[c],
                                        jnp.where(gt2, v, m2s[c])))
                new[3].append(jnp.where(gt1, a1s[c],
                                        jnp.where(gt2, ev, a2s[c])))
                new[0].append(jnp.where(gt1, v, m1s[c]))
                new[1].append(jnp.where(gt1, ev, a1s[c]))
            return tuple(tuple(t) for t in new)

        ninf = jnp.full((L,), -jnp.inf, f32)
        zero_i = jnp.zeros((L,), i32)
        m1s, a1s, m2s, a2s = lax.fori_loop(
            0, E, step,
            ((ninf,) * n_tok_chunks, (zero_i,) * n_tok_chunks,
             (ninf,) * n_tok_chunks, (zero_i,) * n_tok_chunks))

        ones = jnp.ones((L,), f32)
        for c in range(n_tok_chunks):
            sl = pl.ds(c * L, L)
            r = jnp.exp(m2s[c] - m1s[c])
            w1 = 1.0 / (1.0 + r)
            ai_v[0, sl] = a1s[c]
            ai_v[1, sl] = a2s[c]
            wv_v[0, sl] = w1
            wv_v[1, sl] = 1.0 - w1

        for c in range(n_exp_chunks):
            act_v[pl.ds(c * L, L)] = jnp.zeros((L,), f32)
        for c in range(n_tok_chunks):
            plsc.store_scatter(act_v, [a1s[c]], ones)
            plsc.store_scatter(act_v, [a2s[c]], ones)

        carry = f32(0.0)
        last_active = i32(0)
        actives, cums = [], []
        for c in range(n_exp_chunks):
            ch = act_v[pl.ds(c * L, L)]
            cs = plsc.cumsum(ch) + carry
            carry = carry + jnp.sum(ch)
            eid = lane + i32(c * L)
            last_active = jnp.maximum(
                last_active, jnp.max(jnp.where(ch > 0.5, eid, -1)))
            actives.append(ch)
            cums.append(cs)

        lastv = jnp.full((L,), 0, i32) + last_active
        for c in range(n_exp_chunks):
            sched_v[pl.ds(c * L, L)] = lastv
        for c in range(n_exp_chunks):
            rnk = (cums[c] - 1.0).astype(i32)
            eid = lane + i32(c * L)
            plsc.store_scatter(sched_v, [rnk], eid, mask=actives[c] > 0.5)

        nact_v[...] = jnp.full((L,), 0, i32) + carry.astype(i32)

        pltpu.sync_copy(ai_v, ai_hbm)
        pltpu.sync_copy(wv_v, wv_hbm)
        pltpu.sync_copy(sched_v, sched_hbm)
        pltpu.sync_copy(nact_v, nact_hbm)


def _expert_kernel(sched_ref, nact_ref, x_ref, w1_ref, b1_ref, w2_ref, b2_ref,
                   w3_ref, b3_ref, ai_ref, wv_ref, out_ref):
    g = pl.program_id(0)
    T = x_ref.shape[0]
    f32 = jnp.float32

    @pl.when(g == 0)
    def _():
        out_ref[...] = jnp.zeros_like(out_ref)

    @pl.when(g < nact_ref[0])
    def _():
        e = sched_ref[g]
        h1 = jnp.dot(x_ref[...], w1_ref[0], preferred_element_type=f32)
        h1 = h1 + b1_ref[pl.ds(e, 1), :]
        h2 = jnp.dot(h1, w2_ref[0], preferred_element_type=f32)
        h2 = h2 + b2_ref[pl.ds(e, 1), :]
        a = h2 * (1.0 / (1.0 + jnp.exp(-h2)))
        y = jnp.dot(a, w3_ref[0], preferred_element_type=f32)
        y = y + b3_ref[pl.ds(e, 1), :]

        c = (jnp.where(ai_ref[0:1, :] == e, wv_ref[0:1, :], f32(0.0))
             + jnp.where(ai_ref[1:2, :] == e, wv_ref[1:2, :], f32(0.0)))
        ii = lax.broadcasted_iota(jnp.int32, (T, T), 0)
        jj = lax.broadcasted_iota(jnp.int32, (T, T), 1)
        dc = jnp.where(ii == jj, c, f32(0.0))
        out_ref[...] += jnp.dot(dc, y, preferred_element_type=f32)


def kernel(x, Wr, br, W1, b1, W2, b2, W3, b3):
    B, S, D = x.shape
    E = Wr.shape[1]
    H = W1.shape[2]
    T = B * S
    x2 = x.reshape(T, D)

    logits_t = pl.pallas_call(
        _logits_kernel,
        out_shape=jax.ShapeDtypeStruct((E, T), jnp.float32),
    )(x2, Wr, br.reshape(E, 1))

    sc_router = functools.partial(
        pl.kernel,
        out_type=[
            jax.ShapeDtypeStruct((2, T), jnp.int32),
            jax.ShapeDtypeStruct((2, T), jnp.float32),
            jax.ShapeDtypeStruct((E,), jnp.int32),
            jax.ShapeDtypeStruct((16,), jnp.int32),
        ],
        mesh=plsc.VectorSubcoreMesh(core_axis_name="c", subcore_axis_name="s",
                                    num_cores=1),
        compiler_params=pltpu.CompilerParams(needs_layout_passes=False),
        scratch_types=[
            pltpu.VMEM((E, T), jnp.float32),
            pltpu.VMEM((E,), jnp.float32),
            pltpu.VMEM((2, T), jnp.int32),
            pltpu.VMEM((2, T), jnp.float32),
            pltpu.VMEM((E,), jnp.int32),
            pltpu.VMEM((16,), jnp.int32),
        ],
    )(_sc_router_body)
    ai, wv, sched, nactv = sc_router(logits_t)

    out = pl.pallas_call(
        _expert_kernel,
        grid_spec=pltpu.PrefetchScalarGridSpec(
            num_scalar_prefetch=2,
            grid=(E,),
            in_specs=[
                pl.BlockSpec((T, D), lambda g, s, n: (0, 0)),
                pl.BlockSpec((1, D, H), lambda g, s, n: (s[g], 0, 0)),
                pl.BlockSpec((E, H), lambda g, s, n: (0, 0)),
                pl.BlockSpec((1, H, H), lambda g, s, n: (s[g], 0, 0)),
                pl.BlockSpec((E, H), lambda g, s, n: (0, 0)),
                pl.BlockSpec((1, H, D), lambda g, s, n: (s[g], 0, 0)),
                pl.BlockSpec((E, D), lambda g, s, n: (0, 0)),
                pl.BlockSpec((2, T), lambda g, s, n: (0, 0)),
                pl.BlockSpec((2, T), lambda g, s, n: (0, 0)),
            ],
            out_specs=pl.BlockSpec((T, D), lambda g, s, n: (0, 0)),
        ),
        out_shape=jax.ShapeDtypeStruct((T, D), jnp.float32),
        compiler_params=pltpu.CompilerParams(
            dimension_semantics=("arbitrary",),
        ),
    )(sched, nactv, x2, W1, b1, W2, b2, W3, b3, ai, wv)

    return out.reshape(B, S, D)
